# baseline (speedup 1.0000x reference)
import functools

import jax
import jax.numpy as jnp
from jax import lax
from jax.experimental import pallas as pl
from jax.experimental.pallas import tpu as pltpu
from jax.experimental.pallas import tpu_sc as plsc

_IDS = 8
_NB = 2


@functools.cache
def _build(batch: int, hist: int, vocab: int, dim: int):
    info = plsc.get_sparse_core_info()
    nw = info.num_cores * info.num_subcores
    assert batch % (nw * _IDS) == 0
    b_per_w = batch // nw
    nblocks = b_per_w // _IDS

    mesh = plsc.VectorSubcoreMesh(core_axis_name="c", subcore_axis_name="s")

    @functools.partial(
        pl.kernel,
        mesh=mesh,
        out_type=jax.ShapeDtypeStruct((batch * hist, dim), jnp.float32),
        scratch_types=[
            pltpu.VMEM((_NB, _IDS, hist), jnp.int32),
            pltpu.VMEM((_NB, hist, dim), jnp.float32),
            pltpu.SemaphoreType.DMA((_NB,)),
            pltpu.SemaphoreType.DMA((_NB,)),
            pltpu.SemaphoreType.DMA((_NB,)),
        ],
        compiler_params=pltpu.CompilerParams(use_tc_tiling_on_sc=False),
    )
    def gather_kernel(ids_hbm, table_hbm, out_hbm, idx_v, rows_v, isem, gsem, osem):
        wid = lax.axis_index("s") * info.num_cores + lax.axis_index("c")
        base = wid * b_per_w

        def ids_copy(blk, ib):
            return pltpu.make_async_copy(
                ids_hbm.at[pl.ds(base + blk * _IDS, _IDS)],
                idx_v.at[ib], isem.at[ib])

        def gathers(rr, ib, rb):
            cps = []
            off = 0
            while off < hist:
                n = min(128, hist - off)
                cps.append(pltpu.make_async_copy(
                    table_hbm.at[idx_v.at[ib, rr, pl.ds(off, n)]],
                    rows_v.at[rb, pl.ds(off, n)], gsem.at[rb]))
                off += n
            return cps

        def out_copy(r, rb):
            return pltpu.make_async_copy(
                rows_v.at[rb],
                out_hbm.at[pl.ds((base + r) * hist, hist)], osem.at[rb])

        ids_copy(0, 0).start()
        ids_copy(1, 1).start()
        ids_copy(0, 0).wait()
        for cp in gathers(0, 0, 0):
            cp.start()

        def row_body(r, carry):
            rb = lax.rem(r, _NB)
            nrb = lax.rem(r + 1, _NB)
            blk = r // _IDS
            ib = lax.rem(blk, _NB)
            @pl.when(r + 1 < b_per_w)
            def _():
                nblk = (r + 1) // _IDS
                nib = lax.rem(nblk, _NB)
                @pl.when(lax.rem(r + 1, _IDS) == 0)
                def _():
                    ids_copy(nblk, nib).wait()
                @pl.when(r >= 1)
                def _():
                    out_copy(r - 1, nrb).wait()
                for cp in gathers(lax.rem(r + 1, _IDS), nib, nrb):
                    cp.start()
            for cp in gathers(lax.rem(r, _IDS), ib, rb):
                cp.wait()
            out_copy(r, rb).start()
            @pl.when((lax.rem(r, _IDS) == _IDS - 1) & (blk + 2 < nblocks))
            def _():
                ids_copy(blk + 2, ib).start()
            return carry

        lax.fori_loop(0, b_per_w, row_body, 0)
        out_copy(b_per_w - 2, lax.rem(b_per_w - 2, _NB)).wait()
        out_copy(b_per_w - 1, lax.rem(b_per_w - 1, _NB)).wait()

    return gather_kernel


def kernel(input_ids, table):
    batch, hist = input_ids.shape
    vocab, dim = table.shape
    ids = input_ids.astype(jnp.int32)
    out = _build(batch, hist, vocab, dim)(ids, table)
    return out.reshape(batch, hist, dim)


# speedup vs baseline: 1.0045x; 1.0045x over previous
import functools

import jax
import jax.numpy as jnp
from jax import lax
from jax.experimental import pallas as pl
from jax.experimental.pallas import tpu as pltpu
from jax.experimental.pallas import tpu_sc as plsc

_LANE = 128
_R = 5
_NBUF = 2


@functools.cache
def _build(nrows: int, vocab: int, dim: int):
    info = plsc.get_sparse_core_info()
    nw = info.num_cores * info.num_subcores
    assert nrows % nw == 0
    rows_per_w = nrows // nw
    assert rows_per_w % (_R * _NBUF) == 0
    nchunks = rows_per_w // _R

    mesh = plsc.VectorSubcoreMesh(core_axis_name="c", subcore_axis_name="s")

    @functools.partial(
        pl.kernel,
        mesh=mesh,
        out_type=jax.ShapeDtypeStruct((nrows, _LANE, dim), jnp.float32),
        scratch_types=[
            [pltpu.VMEM((_R, _LANE), jnp.int32)] * _NBUF,
            [pltpu.VMEM((_R, _LANE, dim), jnp.float32)] * _NBUF,
            [pltpu.SemaphoreType.DMA] * _NBUF,
            [pltpu.SemaphoreType.DMA] * _NBUF,
            [pltpu.SemaphoreType.DMA] * _NBUF,
        ],
        compiler_params=pltpu.CompilerParams(use_tc_tiling_on_sc=False),
    )
    def gather_kernel(ids_hbm, table_hbm, out_hbm, idx_v, rows_v, isem, gsem, osem):
        wid = lax.axis_index("s") * info.num_cores + lax.axis_index("c")
        base = wid * rows_per_w

        def idx_copy(c, b):
            return pltpu.make_async_copy(
                ids_hbm.at[pl.ds(base + c * _R, _R)], idx_v[b], isem[b])

        def out_copy(c, b):
            return pltpu.make_async_copy(
                rows_v[b], out_hbm.at[pl.ds(base + c * _R, _R)], osem[b])

        idx_copy(0, 0).start()

        def pair_body(cc, carry):
            for b in range(_NBUF):
                c = cc * _NBUF + b
                @pl.when(c >= _NBUF)
                def _():
                    out_copy(c, b).wait()
                idx_copy(c, b).wait()
                @pl.when(c + 1 < nchunks)
                def _():
                    idx_copy(c + 1, (b + 1) % _NBUF).start()
                cps = [
                    pltpu.async_copy(
                        table_hbm.at[idx_v[b].at[j]], rows_v[b].at[j], gsem[b])
                    for j in range(_R)
                ]
                for cp in cps:
                    cp.wait()
                out_copy(c, b).start()
            return carry

        lax.fori_loop(0, nchunks // _NBUF, pair_body, 0)
        for b in range(_NBUF):
            out_copy(nchunks - _NBUF + b, b).wait()

    return gather_kernel


def kernel(input_ids, table):
    batch, hist = input_ids.shape
    vocab, dim = table.shape
    total = batch * hist
    nrows = total // _LANE
    ids = input_ids.reshape(nrows, _LANE).astype(jnp.int32)
    out = _build(nrows, vocab, dim)(ids, table)
    return out.reshape(batch, hist, dim)


# speedup vs baseline: 1.3121x; 1.3062x over previous
import functools

import jax
import jax.numpy as jnp
from jax import lax
from jax.experimental import pallas as pl
from jax.experimental.pallas import tpu as pltpu
from jax.experimental.pallas import tpu_sc as plsc

_IDS = 8
_NB = 2
_RB = 3
_SEG = (128, 72)


@functools.cache
def _build(batch: int, hist: int, vocab: int, dim: int):
    info = plsc.get_sparse_core_info()
    nw = info.num_cores * info.num_subcores
    assert batch % (nw * _IDS) == 0
    assert sum(_SEG) == hist
    b_per_w = batch // nw
    nblocks = b_per_w // _IDS
    nsegs = 2 * b_per_w
    segmax = max(_SEG)

    mesh = plsc.VectorSubcoreMesh(core_axis_name="c", subcore_axis_name="s")

    @functools.partial(
        pl.kernel,
        mesh=mesh,
        out_type=jax.ShapeDtypeStruct((batch, hist, 2 * dim), jnp.float32),
        scratch_types=[
            pltpu.VMEM((_NB, _IDS, hist), jnp.int32),
            pltpu.VMEM((_RB, segmax, 2 * dim), jnp.float32),
            pltpu.SemaphoreType.DMA((_NB,)),
            pltpu.SemaphoreType.DMA((_RB,)),
            pltpu.SemaphoreType.DMA((_RB,)),
        ],
        compiler_params=pltpu.CompilerParams(use_tc_tiling_on_sc=True),
    )
    def gather_kernel(ids_hbm, table_hbm, out_hbm, idx_v, rows_v,
                      isem, gsem, osem):
        wid = lax.axis_index("s") * info.num_cores + lax.axis_index("c")
        base = wid * b_per_w

        def ids_copy(blk, ib):
            return pltpu.make_async_copy(
                ids_hbm.at[pl.ds(base + blk * _IDS, _IDS)],
                idx_v.at[ib], isem.at[ib])

        def gather_seg(r, s, gb):
            off = sum(_SEG[:s])
            n = _SEG[s]
            ib = lax.rem(r // _IDS, _NB)
            rr = lax.rem(r, _IDS)
            return pltpu.make_async_copy(
                table_hbm.at[idx_v.at[ib, rr, pl.ds(off, n)]],
                rows_v.at[gb, pl.ds(0, n)], gsem.at[gb])

        def out_seg(r, s, gb):
            off = sum(_SEG[:s])
            n = _SEG[s]
            return pltpu.make_async_copy(
                rows_v.at[gb, pl.ds(0, n)],
                out_hbm.at[base + r, pl.ds(off, n)], osem.at[gb])

        ids_copy(0, 0).start()
        ids_copy(1, 1).start()
        ids_copy(0, 0).wait()
        gather_seg(0, 0, 0).start()

        def seg_stuff(g, s):
            r = g // 2
            gb = lax.rem(g, _RB)
            ngb = lax.rem(g + 1, _RB)
            @pl.when(g >= 2)
            def _():
                out_seg(r - 1, s, ngb).wait()
            @pl.when(g + 1 < nsegs)
            def _():
                nr = r + (1 if s == 1 else 0)
                ns = (s + 1) % 2
                if s == 1:
                    @pl.when(lax.rem(nr, _IDS) == 0)
                    def _():
                        ids_copy(nr // _IDS, lax.rem(nr // _IDS, _NB)).wait()
                gather_seg(nr, ns, ngb).start()
            gather_seg(r, s, gb).wait()
            out_seg(r, s, gb).start()
            if s == 1:
                blk = r // _IDS
                @pl.when((lax.rem(r, _IDS) == _IDS - 1)
                         & (blk + 2 < nblocks))
                def _():
                    ids_copy(blk + 2, lax.rem(blk, _NB)).start()

        def seg_body(g, carry):
            @pl.when(lax.rem(g, 2) == 0)
            def _():
                seg_stuff(g, 0)
            @pl.when(lax.rem(g, 2) == 1)
            def _():
                seg_stuff(g, 1)
            return carry

        lax.fori_loop(0, nsegs, seg_body, 0)
        out_seg(b_per_w - 1, 0, lax.rem(nsegs - 2, _RB)).wait()
        out_seg(b_per_w - 1, 1, lax.rem(nsegs - 1, _RB)).wait()

    return gather_kernel


def kernel(input_ids, table):
    batch, hist = input_ids.shape
    vocab, dim = table.shape
    ids = input_ids.astype(jnp.int32)
    table_pad = jnp.pad(table, ((0, 0), (0, dim)))
    out = _build(batch, hist, vocab, dim)(ids, table_pad)
    return out[:, :, :dim]
